# baseline (speedup 1.0000x reference)
import functools

import jax
import jax.numpy as jnp
from jax import lax
from jax.experimental import pallas as pl
from jax.experimental.pallas import tpu as pltpu
from jax.experimental.pallas import tpu_sc as plsc

_SCALE = 12.0
_NUM_CORES = 2
_NUM_SUBCORES = 16
_NW = _NUM_CORES * _NUM_SUBCORES
_L = 16
_CB = 128
_OP = 129


def _body(seq, batch, d, ids_hbm, pairs_hbm, out_hbm,
          idv, pv, gv, ov, gsem, ssem):
    wid = lax.axis_index("s") * _NUM_CORES + lax.axis_index("c")
    n_chunks_b = batch // _CB
    n_chunks = seq * n_chunks_b
    per_w = n_chunks // _NW

    iota = lax.iota(jnp.int32, _L)

    def chunk_of(k):
        c = k * _NW + wid
        return c // n_chunks_b, c % n_chunks_b

    def fetch(k, slot):
        s, bc = chunk_of(k)
        return pltpu.make_async_copy(
            ids_hbm.at[s, pl.ds(bc * _CB, _CB)], idv.at[slot], gsem.at[slot])

    def gather(slot):
        return pltpu.make_async_copy(
            pairs_hbm.at[pv.at[slot]], gv.at[slot], gsem.at[slot])

    def store(k, slot):
        s, bc = chunk_of(k)
        return pltpu.make_async_copy(
            ov.at[slot, :, pl.ds(0, _CB)],
            out_hbm.at[s, :, pl.ds(bc * _CB, _CB)], ssem.at[slot])

    def prep_idx(slot):
        for i in range(_CB // _L):
            sl = pl.ds(i * _L, _L)
            ids = idv[slot, sl]
            pv[slot, sl] = lax.shift_right_logical(ids, 1)
            idv[slot, sl] = (ids & 1) * d

    def transpose_scale(slot):
        def row_body(r, carry):
            rvec = jnp.full((_L,), 0, jnp.int32) + r
            off_b = plsc.load_gather(idv.at[slot], [rvec])
            mask = off_b > 0
            for f0 in range(0, d, _L):
                lo = gv[slot, r, pl.ds(f0, _L)]
                hi = gv[slot, r, pl.ds(d + f0, _L)]
                vals = jnp.where(mask, hi, lo)
                plsc.store_scatter(
                    ov.at[slot], [f0 + iota, rvec], vals * _SCALE)
            return carry

        lax.fori_loop(0, _CB, row_body, 0, unroll=8)

    fetch(0, 0).start()
    fetch(1, 1).start()
    fetch(0, 0).wait()
    prep_idx(0)
    gather(0).start()

    def loop(k, carry):
        slot = lax.rem(k, 2)
        nslot = 1 - slot
        @pl.when(k + 1 < per_w)
        def _():
            fetch(k + 1, nslot).wait()
            prep_idx(nslot)
            gather(nslot).start()
        gather(slot).wait()

        @pl.when(k >= 2)
        def _():
            store(k - 2, slot).wait()
        transpose_scale(slot)
        store(k, slot).start()

        @pl.when(k + 2 < per_w)
        def _():
            fetch(k + 2, slot).start()
        return carry

    lax.fori_loop(0, per_w, loop, 0)
    store(per_w - 2, 0 if per_w % 2 == 0 else 1).wait()
    store(per_w - 1, 1 if per_w % 2 == 0 else 0).wait()


@jax.jit
def kernel(input_ids, weight):
    b, s = input_ids.shape
    v, d = weight.shape
    ids_t = input_ids.T
    pairs = weight.reshape(v // 2, 2 * d)

    mesh = plsc.VectorSubcoreMesh(core_axis_name="c", subcore_axis_name="s")
    run = functools.partial(
        pl.kernel,
        mesh=mesh,
        out_type=jax.ShapeDtypeStruct((s, d, b), jnp.float32),
        scratch_types=[
            pltpu.VMEM((2, _CB), jnp.int32),
            pltpu.VMEM((2, _CB), jnp.int32),
            pltpu.VMEM((2, _CB, 2 * d), jnp.float32),
            pltpu.VMEM((2, d, _OP), jnp.float32),
            pltpu.SemaphoreType.DMA((2,)),
            pltpu.SemaphoreType.DMA((2,)),
        ],
        compiler_params=pltpu.CompilerParams(needs_layout_passes=False),
    )(functools.partial(_body, s, b, d))
    out = run(ids_t, pairs)
    return jnp.transpose(out, (2, 0, 1))


# speedup vs baseline: 1.4277x; 1.4277x over previous
import functools

import jax
import jax.numpy as jnp
from jax import lax
from jax.experimental import pallas as pl
from jax.experimental.pallas import tpu as pltpu
from jax.experimental.pallas import tpu_sc as plsc

_SCALE = 12.0
_NUM_CORES = 2
_NUM_SUBCORES = 16
_NW = _NUM_CORES * _NUM_SUBCORES
_LANES = 16
_CHUNK = 128
_NBUF = 4


def _body(n_rows, d, ids_hbm, table_hbm, out_hbm, idx_v, in_v, out_v, gsem, ssem):
    per_w = n_rows // _NW
    wid = lax.axis_index("s") * _NUM_CORES + lax.axis_index("c")
    base = wid * per_w
    pltpu.sync_copy(ids_hbm.at[pl.ds(base, per_w)], idx_v)
    n_chunks = per_w // _CHUNK

    def gather(g, b):
        return pltpu.make_async_copy(
            table_hbm.at[idx_v.at[pl.ds(g * _CHUNK, _CHUNK)]],
            in_v.at[b],
            gsem.at[b],
        )

    def store(g, b):
        return pltpu.make_async_copy(
            out_v.at[b],
            out_hbm.at[pl.ds(base + g * _CHUNK, _CHUNK)],
            ssem.at[b],
        )

    for b in range(_NBUF):
        gather(b, b).start()

    def outer_body(o, carry):
        for b in range(_NBUF):
            g = o * _NBUF + b
            gather(g, b).wait()

            @pl.when(g >= _NBUF)
            def _():
                store(g - _NBUF, b).wait()

            def scale_rows(r, c2):
                for u in range(4):
                    for j in range(d // _LANES):
                        sl = pl.ds(j * _LANES, _LANES)
                        out_v[b, r * 4 + u, sl] = in_v[b, r * 4 + u, sl] * _SCALE
                return c2

            lax.fori_loop(0, _CHUNK // 4, scale_rows, 0, unroll=2)

            @pl.when(g + _NBUF < n_chunks)
            def _():
                gather(g + _NBUF, b).start()

            store(g, b).start()
        return carry

    lax.fori_loop(0, n_chunks // _NBUF, outer_body, 0)

    for b in range(_NBUF):
        store(n_chunks - _NBUF + b, b).wait()


@jax.jit
def kernel(input_ids, weight):
    b, s = input_ids.shape
    v, d = weight.shape
    n_rows = b * s
    flat_ids = input_ids.reshape(n_rows)

    mesh = plsc.VectorSubcoreMesh(core_axis_name="c", subcore_axis_name="s")
    per_w = n_rows // _NW
    run = functools.partial(
        pl.kernel,
        mesh=mesh,
        out_type=jax.ShapeDtypeStruct((n_rows, d), jnp.float32),
        scratch_types=[
            pltpu.VMEM((per_w,), jnp.int32),
            pltpu.VMEM((_NBUF, _CHUNK, d), jnp.float32),
            pltpu.VMEM((_NBUF, _CHUNK, d), jnp.float32),
            pltpu.SemaphoreType.DMA((_NBUF,)),
            pltpu.SemaphoreType.DMA((_NBUF,)),
        ],
        compiler_params=pltpu.CompilerParams(use_tc_tiling_on_sc=False),
    )(functools.partial(_body, n_rows, d))
    out = run(flat_ids, weight)
    return out.reshape(b, s, d)
